# baseline (speedup 1.0000x reference)
import functools

import jax
import jax.numpy as jnp
from jax import lax
from jax.experimental import pallas as pl
from jax.experimental.pallas import tpu as pltpu
from jax.experimental.pallas import tpu_sc as plsc

_F = 26
_B = 4096
_CHUNK = 128


@functools.cache
def _build(V, D, N):
    info = plsc.get_sparse_core_info()
    NC, NS, L = info.num_cores, info.num_subcores, info.num_lanes
    NW = NC * NS
    assert N % (NW * _CHUNK) == 0
    rpw = N // NW
    n_chunks = rpw // _CHUNK
    mesh = plsc.VectorSubcoreMesh(core_axis_name="c", subcore_axis_name="s")

    @functools.partial(
        pl.kernel,
        out_type=(
            jax.ShapeDtypeStruct((N, D), jnp.float32),
            jax.ShapeDtypeStruct((_F * L,), jnp.int32),
        ),
        mesh=mesh,
        compiler_params=pltpu.CompilerParams(use_tc_tiling_on_sc=False),
        scratch_types=[
            pltpu.VMEM((rpw,), jnp.int32),
            pltpu.VMEM((rpw, D), jnp.float32),
            pltpu.VMEM((_B,), jnp.int32),
            pltpu.VMEM((L,), jnp.int32),
            pltpu.SemaphoreType.DMA,
        ],
    )
    def gather_kernel(table_hbm, values_hbm, lengths_hbm, out_hbm, sums_hbm,
                      idx_v, rows_v, len_v, acc_v, gsem):
        wid = lax.axis_index("s") * NC + lax.axis_index("c")
        base = wid * rpw
        pltpu.sync_copy(values_hbm.at[pl.ds(base, rpw)], idx_v)
        for j in range(n_chunks):
            pltpu.async_copy(table_hbm.at[idx_v.at[pl.ds(j * _CHUNK, _CHUNK)]],
                             rows_v.at[pl.ds(j * _CHUNK, _CHUNK)], gsem)

        @pl.when(wid < _F)
        def _():
            pltpu.sync_copy(lengths_hbm.at[pl.ds(wid * _B, _B)], len_v)

            def step(i, acc):
                return acc + len_v[pl.ds(i * L, L)]

            acc_v[...] = lax.fori_loop(0, _B // L, step,
                                       jnp.zeros((L,), jnp.int32))
            pltpu.sync_copy(acc_v, sums_hbm.at[pl.ds(wid * L, L)])

        pltpu.make_async_copy(out_hbm.at[pl.ds(base, rpw)], rows_v,
                              gsem).wait()
        pltpu.sync_copy(rows_v, out_hbm.at[pl.ds(base, rpw)])

    return gather_kernel


@jax.jit
def kernel(table, values, lengths):
    V, D = table.shape
    N = values.shape[0]
    L = 16
    rows, sums = _build(V, D, N)(table, values, lengths)
    split_embeddings = rows.reshape(_F, _B, D)
    reduce_lengths = sums.reshape(_F, L).sum(axis=1)
    offsets = jnp.concatenate([
        jnp.zeros((1,), reduce_lengths.dtype),
        jnp.cumsum(reduce_lengths),
    ])
    split_lengths = lengths.reshape(_F, _B)
    return split_embeddings, split_lengths, offsets
